# baseline (speedup 1.0000x reference)
import functools

import jax
import jax.numpy as jnp
from jax import lax
from jax.experimental import pallas as pl
from jax.experimental.pallas import tpu as pltpu
from jax.experimental.pallas import tpu_sc as plsc

EMB = 64
HID = 128
EPS = 1e-12

_NC = 2
_NS = 16
_NW = _NC * _NS

_IDXW = 128
_K = 5


def _tc_project(table_t, w_half):
  V = table_t.shape[1]
  T = 8192 if V > 8192 else V
  grid = (pl.cdiv(V, T),)

  def body(t_ref, w_ref, o_ref):
    o_ref[...] = lax.dot_general(
        t_ref[...], w_ref[...], (((0,), (1,)), ((), ())),
        preferred_element_type=jnp.float32).astype(jnp.bfloat16)

  return pl.pallas_call(
      body,
      grid=grid,
      in_specs=[
          pl.BlockSpec((EMB, T), lambda i: (0, i)),
          pl.BlockSpec((HID, EMB), lambda i: (0, 0)),
      ],
      out_specs=pl.BlockSpec((T, HID), lambda i: (i, 0)),
      out_shape=jax.ShapeDtypeStruct((V, HID), jnp.bfloat16),
  )(table_t, w_half)


def _sc_gather_add(ids2d, cats2d, idproj, catproj):
  NR = ids2d.shape[0]
  rpw = NR // _NW

  mesh = plsc.VectorSubcoreMesh(core_axis_name="c", subcore_axis_name="s")

  @functools.partial(
      pl.kernel,
      mesh=mesh,
      compiler_params=pltpu.CompilerParams(use_tc_tiling_on_sc=False),
      out_type=jax.ShapeDtypeStruct((NR, _IDXW, HID), jnp.bfloat16),
      scratch_types=[
          pltpu.VMEM((_K, _IDXW), jnp.int32),
          pltpu.VMEM((_K, _IDXW), jnp.int32),
          pltpu.VMEM((_K, _IDXW, HID), jnp.bfloat16),
          pltpu.SemaphoreType.DMA,
          pltpu.SemaphoreType.DMA,
          pltpu.SemaphoreType.DMA,
      ],
  )
  def k(ids_hbm, cats_hbm, idp_hbm, catp_hbm, out,
        idx_i, idx_c, rows, sem_g, sem_a, sem_w):
    wid = lax.axis_index("s") * _NC + lax.axis_index("c")
    base = wid * rpw

    def body(g, carry):
      rb = base + g * _K
      pltpu.sync_copy(ids_hbm.at[pl.ds(rb, _K)], idx_i)
      pltpu.sync_copy(cats_hbm.at[pl.ds(rb, _K)], idx_c)
      gcp = [pltpu.async_copy(idp_hbm.at[idx_i.at[j]], rows.at[j], sem_g)
             for j in range(_K)]
      acp = [None] * _K
      wcp = [None] * _K
      for j in range(_K):
        gcp[j].wait()
        acp[j] = pltpu.async_copy(catp_hbm.at[idx_c.at[j]], rows.at[j],
                                  sem_a, add=True)
      for j in range(_K):
        acp[j].wait()
        wcp[j] = pltpu.async_copy(rows.at[j], out.at[rb + j], sem_w)
      for j in range(_K):
        wcp[j].wait()
      return carry

    lax.fori_loop(0, rpw // _K, body, 0)

  return k(ids2d, cats2d, idproj, catproj)


def _tc_ln(h, b, gamma, beta):
  N = h.shape[0]
  T = 8192
  grid = (N // T,)

  def body(h_ref, b_ref, g_ref, bt_ref, o_ref):
    x = h_ref[...].astype(jnp.float32) + b_ref[...]
    mu = jnp.mean(x, axis=-1, keepdims=True)
    d = x - mu
    var = jnp.mean(d * d, axis=-1, keepdims=True)
    o_ref[...] = d * lax.rsqrt(var + EPS) * g_ref[...] + bt_ref[...]

  return pl.pallas_call(
      body,
      grid=grid,
      in_specs=[
          pl.BlockSpec((T, HID), lambda i: (i, 0)),
          pl.BlockSpec((1, HID), lambda i: (0, 0)),
          pl.BlockSpec((1, HID), lambda i: (0, 0)),
          pl.BlockSpec((1, HID), lambda i: (0, 0)),
      ],
      out_specs=pl.BlockSpec((T, HID), lambda i: (i, 0)),
      out_shape=jax.ShapeDtypeStruct((N, HID), jnp.float32),
  )(h, b, gamma, beta)


def kernel(input_ids, category_ids, id_table, cat_table, W, b, gamma, beta):
  B, L = input_ids.shape
  N = B * L
  NR = N // _IDXW

  ids2d = input_ids.reshape(NR, _IDXW).astype(jnp.int32)
  cats2d = category_ids.reshape(NR, _IDXW).astype(jnp.int32)

  idproj = _tc_project(id_table.T, W[:, :EMB])
  catproj = _tc_project(cat_table.T, W[:, EMB:])

  h = _sc_gather_add(ids2d, cats2d, idproj, catproj)

  out = _tc_ln(h.reshape(N, HID), b.reshape(1, HID),
               gamma.reshape(1, HID), beta.reshape(1, HID))
  return out.reshape(B, L, HID)


# speedup vs baseline: 2.2751x; 2.2751x over previous
import functools

import jax
import jax.numpy as jnp
from jax import lax
from jax.experimental import pallas as pl
from jax.experimental.pallas import tpu as pltpu
from jax.experimental.pallas import tpu_sc as plsc

EMB = 64
HID = 128
EPS = 1e-12

_NC = 2
_NS = 16
_NW = _NC * _NS

_IDXW = 128
_K = 5


def _tc_project(table_t, w_half):
  V = table_t.shape[1]
  T = 8192 if V > 8192 else V
  grid = (pl.cdiv(V, T),)

  def body(t_ref, w_ref, o_ref):
    o_ref[...] = lax.dot_general(
        t_ref[...], w_ref[...], (((0,), (1,)), ((), ())),
        preferred_element_type=jnp.float32)

  return pl.pallas_call(
      body,
      grid=grid,
      in_specs=[
          pl.BlockSpec((EMB, T), lambda i: (0, i)),
          pl.BlockSpec((HID, EMB), lambda i: (0, 0)),
      ],
      out_specs=pl.BlockSpec((T, HID), lambda i: (i, 0)),
      out_shape=jax.ShapeDtypeStruct((V, HID), jnp.float32),
  )(table_t, w_half)


def _sc_gather_add(ids2d, cats2d, idproj, catproj):
  NR = ids2d.shape[0]
  rpw = NR // _NW

  mesh = plsc.VectorSubcoreMesh(core_axis_name="c", subcore_axis_name="s")

  @functools.partial(
      pl.kernel,
      mesh=mesh,
      compiler_params=pltpu.CompilerParams(use_tc_tiling_on_sc=False),
      out_type=jax.ShapeDtypeStruct((NR, _IDXW, HID), jnp.float32),
      scratch_types=[
          pltpu.VMEM((_K, _IDXW), jnp.int32),
          pltpu.VMEM((_K, _IDXW), jnp.int32),
          pltpu.VMEM((_K, _IDXW, HID), jnp.float32),
          pltpu.SemaphoreType.DMA,
          pltpu.SemaphoreType.DMA,
          pltpu.SemaphoreType.DMA,
      ],
  )
  def k(ids_hbm, cats_hbm, idp_hbm, catp_hbm, out,
        idx_i, idx_c, rows, sem_g, sem_a, sem_w):
    wid = lax.axis_index("s") * _NC + lax.axis_index("c")
    base = wid * rpw

    def body(g, carry):
      rb = base + g * _K
      pltpu.sync_copy(ids_hbm.at[pl.ds(rb, _K)], idx_i)
      pltpu.sync_copy(cats_hbm.at[pl.ds(rb, _K)], idx_c)
      gcp = [pltpu.async_copy(idp_hbm.at[idx_i.at[j]], rows.at[j], sem_g)
             for j in range(_K)]
      acp = [None] * _K
      wcp = [None] * _K
      for j in range(_K):
        gcp[j].wait()
        acp[j] = pltpu.async_copy(catp_hbm.at[idx_c.at[j]], rows.at[j],
                                  sem_a, add=True)
      for j in range(_K):
        acp[j].wait()
        wcp[j] = pltpu.async_copy(rows.at[j], out.at[rb + j], sem_w)
      for j in range(_K):
        wcp[j].wait()
      return carry

    lax.fori_loop(0, rpw // _K, body, 0)

  return k(ids2d, cats2d, idproj, catproj)


def _tc_ln_phase(h, b, gamma, beta, prev, s, N):
  Ns = h.shape[0]
  T = 8192
  steps = Ns // T
  base = s * steps

  def body(*refs):
    if s > 0:
      _, h_ref, b_ref, g_ref, bt_ref, o_ref = refs
    else:
      h_ref, b_ref, g_ref, bt_ref, o_ref = refs
    x = h_ref[...] + b_ref[...]
    mu = jnp.mean(x, axis=-1, keepdims=True)
    d = x - mu
    var = jnp.mean(d * d, axis=-1, keepdims=True)
    o_ref[...] = d * lax.rsqrt(var + EPS) * g_ref[...] + bt_ref[...]

  in_specs = [
      pl.BlockSpec((T, HID), lambda i: (i, 0)),
      pl.BlockSpec((1, HID), lambda i: (0, 0)),
      pl.BlockSpec((1, HID), lambda i: (0, 0)),
      pl.BlockSpec((1, HID), lambda i: (0, 0)),
  ]
  args = [h, b, gamma, beta]
  aliases = {}
  if s > 0:
    in_specs = [pl.BlockSpec((8, HID), lambda i: (0, 0))] + in_specs
    args = [prev] + args
    aliases = {0: 0}

  return pl.pallas_call(
      body,
      grid=(steps,),
      in_specs=in_specs,
      out_specs=pl.BlockSpec((T, HID), lambda i: (base + i, 0)),
      out_shape=jax.ShapeDtypeStruct((N, HID), jnp.float32),
      input_output_aliases=aliases,
  )(*args)


def kernel(input_ids, category_ids, id_table, cat_table, W, b, gamma, beta):
  B, L = input_ids.shape
  N = B * L
  NR = N // _IDXW

  ids2d = input_ids.reshape(NR, _IDXW).astype(jnp.int32)
  cats2d = category_ids.reshape(NR, _IDXW).astype(jnp.int32)

  idproj = _tc_project(id_table.T, W[:, :EMB])
  catproj = _tc_project(cat_table.T, W[:, EMB:])

  S = 4
  NRs = NR // S
  b2 = b.reshape(1, HID)
  g2 = gamma.reshape(1, HID)
  bt2 = beta.reshape(1, HID)
  out = None
  for s in range(S):
    h_s = _sc_gather_add(
        lax.slice_in_dim(ids2d, s * NRs, (s + 1) * NRs),
        lax.slice_in_dim(cats2d, s * NRs, (s + 1) * NRs),
        idproj, catproj)
    out = _tc_ln_phase(h_s.reshape(NRs * _IDXW, HID), b2, g2, bt2,
                       out, s, N)
  return out.reshape(B, L, HID)
